# baseline (speedup 1.0000x reference)
import functools

import jax
import jax.numpy as jnp
from jax import lax
from jax.experimental import pallas as pl
from jax.experimental.pallas import tpu as pltpu
from jax.experimental.pallas import tpu_sc as plsc

N = 4096
M = 4096
MS = 2048
MT = M - MS
NC = 2
NS = 16
L = 16
NW = NC * NS
SRC_PER_W = N // NW
NV = SRC_PER_W // L
MG = MS // L
CH = 32

_mesh = plsc.VectorSubcoreMesh(core_axis_name="c", subcore_axis_name="s")


@functools.partial(
    pl.kernel,
    mesh=_mesh,
    out_type=jax.ShapeDtypeStruct((NW, SRC_PER_W), jnp.float32),
    scratch_types=[
        pltpu.VMEM((SRC_PER_W,), jnp.float32),
        pltpu.VMEM((SRC_PER_W,), jnp.float32),
        pltpu.VMEM((SRC_PER_W,), jnp.float32),
        pltpu.VMEM((MS,), jnp.float32),
        pltpu.VMEM((MS,), jnp.float32),
        pltpu.VMEM((MS,), jnp.float32),
        pltpu.VMEM((MS,), jnp.float32),
        pltpu.VMEM((CH * L,), jnp.float32),
        pltpu.VMEM((CH * L,), jnp.float32),
        pltpu.VMEM((CH * L,), jnp.float32),
        pltpu.VMEM((CH * L,), jnp.float32),
        pltpu.VMEM((SRC_PER_W,), jnp.float32),
    ],
)
def _nn_partials_sc(sx_hbm, sy_hbm, sz_hbm, tx_hbm, ty_hbm, tz_hbm, out_hbm,
                    sx_v, sy_v, sz_v, a_v, b_v, c_v, d_v,
                    sa_v, sb_v, sc_v, sd_v, out_v):
    wid = lax.axis_index("s") * NC + lax.axis_index("c")
    base = wid * SRC_PER_W

    pltpu.sync_copy(sx_hbm.at[pl.ds(base, SRC_PER_W)], sx_v)
    pltpu.sync_copy(sy_hbm.at[pl.ds(base, SRC_PER_W)], sy_v)
    pltpu.sync_copy(sz_hbm.at[pl.ds(base, SRC_PER_W)], sz_v)
    pltpu.sync_copy(tx_hbm, a_v)
    pltpu.sync_copy(ty_hbm, b_v)
    pltpu.sync_copy(tz_hbm, c_v)

    def prep(i, carry):
        tx = a_v[pl.ds(i * L, L)]
        ty = b_v[pl.ds(i * L, L)]
        tz = c_v[pl.ds(i * L, L)]
        d_v[pl.ds(i * L, L)] = tx * tx + ty * ty + tz * tz
        a_v[pl.ds(i * L, L)] = tx * jnp.float32(-2.0)
        b_v[pl.ds(i * L, L)] = ty * jnp.float32(-2.0)
        c_v[pl.ds(i * L, L)] = tz * jnp.float32(-2.0)
        return carry

    lax.fori_loop(0, MG, prep, jnp.int32(0), unroll=False)

    big = jnp.full((L,), 1e30, dtype=jnp.float32)
    sxs = [sx_v[pl.ds(k * L, L)] for k in range(NV)]
    sys_ = [sy_v[pl.ds(k * L, L)] for k in range(NV)]
    szs = [sz_v[pl.ds(k * L, L)] for k in range(NV)]

    def chunk(c, accs):
        accs = list(accs)
        tbase = c * CH
        for g in range(CH // L):
            av = a_v[pl.ds(tbase + g * L, L)]
            bv = b_v[pl.ds(tbase + g * L, L)]
            cv = c_v[pl.ds(tbase + g * L, L)]
            dv = d_v[pl.ds(tbase + g * L, L)]
            for l in range(L):
                t = g * L + l
                sa_v[pl.ds(t * L, L)] = jnp.broadcast_to(av[l], (L,))
                sb_v[pl.ds(t * L, L)] = jnp.broadcast_to(bv[l], (L,))
                sc_v[pl.ds(t * L, L)] = jnp.broadcast_to(cv[l], (L,))
                sd_v[pl.ds(t * L, L)] = jnp.broadcast_to(dv[l], (L,))
        for t in range(CH):
            sa = sa_v[pl.ds(t * L, L)]
            sb = sb_v[pl.ds(t * L, L)]
            sc = sc_v[pl.ds(t * L, L)]
            sd = sd_v[pl.ds(t * L, L)]
            for k in range(NV):
                tt = sa * sxs[k] + (sb * sys_[k] + (sc * szs[k] + sd))
                accs[k] = jnp.minimum(accs[k], tt)
        return tuple(accs)

    accs = lax.fori_loop(0, MS // CH, chunk, tuple([big] * NV), unroll=False)

    for k in range(NV):
        s2 = sxs[k] * sxs[k] + sys_[k] * sys_[k] + szs[k] * szs[k]
        out_v[pl.ds(k * L, L)] = accs[k] + s2
    pltpu.sync_copy(out_v, out_hbm.at[wid])


ST_TC = 512
TT_TC = 1024
KP = 128


def _tc_body(s_ref, t_ref, o_ref):
    ti = pl.program_id(1)
    s = s_ref[...]
    t = t_ref[...]
    g = jax.lax.dot_general(
        s, t, (((1,), (1,)), ((), ())),
        preferred_element_type=jnp.float32)
    s2 = jnp.sum(s * s, axis=1, keepdims=True)
    t2 = jnp.sum(t * t, axis=1, keepdims=True)
    d2 = s2 + (t2.T - jnp.float32(2.0) * g)
    m = jnp.min(d2, axis=1, keepdims=True)

    @pl.when(ti == 0)
    def _init():
        o_ref[...] = m

    @pl.when(ti != 0)
    def _acc():
        o_ref[...] = jnp.minimum(o_ref[...], m)


_nn_partials_tc = pl.pallas_call(
    _tc_body,
    grid=(N // ST_TC, MT // TT_TC),
    in_specs=[
        pl.BlockSpec((ST_TC, KP), lambda si, ti: (si, 0)),
        pl.BlockSpec((TT_TC, KP), lambda si, ti: (ti, 0)),
    ],
    out_specs=pl.BlockSpec((ST_TC, 1), lambda si, ti: (si, 0)),
    out_shape=jax.ShapeDtypeStruct((N, 1), jnp.float32),
    compiler_params=pltpu.CompilerParams(
        dimension_semantics=("parallel", "arbitrary")),
)


def kernel(source, target):
    st = source.T
    tt = target.T
    sc_out = _nn_partials_sc(st[0], st[1], st[2],
                             tt[0, :MS], tt[1, :MS], tt[2, :MS])
    s_pad = jnp.pad(source, ((0, 0), (0, KP - 3)))
    t_pad = jnp.pad(target[MS:], ((0, 0), (0, KP - 3)))
    tc_out = _nn_partials_tc(s_pad, t_pad)
    mind = jnp.minimum(sc_out.reshape(N), tc_out.reshape(N))
    rmse = jnp.sqrt(jnp.sum(mind) / jnp.float32(N))
    return jnp.where(rmse < jnp.float32(0.1), jnp.float32(1.0),
                     jnp.float32(0.0))


# speedup vs baseline: 1.1194x; 1.1194x over previous
import functools

import jax
import jax.numpy as jnp
from jax import lax
from jax.experimental import pallas as pl
from jax.experimental.pallas import tpu as pltpu
from jax.experimental.pallas import tpu_sc as plsc

N = 4096
M = 4096
MS = 1536
MT = M - MS
NC = 2
NS = 16
L = 16
NW = NC * NS
SRC_PER_W = N // NW
NV = SRC_PER_W // L
MG = MS // L
CH = 32

_mesh = plsc.VectorSubcoreMesh(core_axis_name="c", subcore_axis_name="s")


@functools.partial(
    pl.kernel,
    mesh=_mesh,
    out_type=jax.ShapeDtypeStruct((NW, SRC_PER_W), jnp.float32),
    scratch_types=[
        pltpu.VMEM((SRC_PER_W,), jnp.float32),
        pltpu.VMEM((SRC_PER_W,), jnp.float32),
        pltpu.VMEM((SRC_PER_W,), jnp.float32),
        pltpu.VMEM((MS,), jnp.float32),
        pltpu.VMEM((MS,), jnp.float32),
        pltpu.VMEM((MS,), jnp.float32),
        pltpu.VMEM((MS,), jnp.float32),
        pltpu.VMEM((CH * L,), jnp.float32),
        pltpu.VMEM((CH * L,), jnp.float32),
        pltpu.VMEM((CH * L,), jnp.float32),
        pltpu.VMEM((CH * L,), jnp.float32),
        pltpu.VMEM((SRC_PER_W,), jnp.float32),
    ],
)
def _nn_partials_sc(sx_hbm, sy_hbm, sz_hbm, tx_hbm, ty_hbm, tz_hbm, out_hbm,
                    sx_v, sy_v, sz_v, a_v, b_v, c_v, d_v,
                    sa_v, sb_v, sc_v, sd_v, out_v):
    wid = lax.axis_index("s") * NC + lax.axis_index("c")
    base = wid * SRC_PER_W

    pltpu.sync_copy(sx_hbm.at[pl.ds(base, SRC_PER_W)], sx_v)
    pltpu.sync_copy(sy_hbm.at[pl.ds(base, SRC_PER_W)], sy_v)
    pltpu.sync_copy(sz_hbm.at[pl.ds(base, SRC_PER_W)], sz_v)
    pltpu.sync_copy(tx_hbm, a_v)
    pltpu.sync_copy(ty_hbm, b_v)
    pltpu.sync_copy(tz_hbm, c_v)

    def prep(i, carry):
        tx = a_v[pl.ds(i * L, L)]
        ty = b_v[pl.ds(i * L, L)]
        tz = c_v[pl.ds(i * L, L)]
        d_v[pl.ds(i * L, L)] = tx * tx + ty * ty + tz * tz
        a_v[pl.ds(i * L, L)] = tx * jnp.float32(-2.0)
        b_v[pl.ds(i * L, L)] = ty * jnp.float32(-2.0)
        c_v[pl.ds(i * L, L)] = tz * jnp.float32(-2.0)
        return carry

    lax.fori_loop(0, MG, prep, jnp.int32(0), unroll=False)

    big = jnp.full((L,), 1e30, dtype=jnp.float32)
    sxs = [sx_v[pl.ds(k * L, L)] for k in range(NV)]
    sys_ = [sy_v[pl.ds(k * L, L)] for k in range(NV)]
    szs = [sz_v[pl.ds(k * L, L)] for k in range(NV)]

    def chunk(c, accs):
        accs = list(accs)
        tbase = c * CH
        for g in range(CH // L):
            av = a_v[pl.ds(tbase + g * L, L)]
            bv = b_v[pl.ds(tbase + g * L, L)]
            cv = c_v[pl.ds(tbase + g * L, L)]
            dv = d_v[pl.ds(tbase + g * L, L)]
            for l in range(L):
                t = g * L + l
                sa_v[pl.ds(t * L, L)] = jnp.broadcast_to(av[l], (L,))
                sb_v[pl.ds(t * L, L)] = jnp.broadcast_to(bv[l], (L,))
                sc_v[pl.ds(t * L, L)] = jnp.broadcast_to(cv[l], (L,))
                sd_v[pl.ds(t * L, L)] = jnp.broadcast_to(dv[l], (L,))
        for t in range(CH):
            sa = sa_v[pl.ds(t * L, L)]
            sb = sb_v[pl.ds(t * L, L)]
            sc = sc_v[pl.ds(t * L, L)]
            sd = sd_v[pl.ds(t * L, L)]
            for k in range(NV):
                tt = sa * sxs[k] + (sb * sys_[k] + (sc * szs[k] + sd))
                accs[k] = jnp.minimum(accs[k], tt)
        return tuple(accs)

    accs = lax.fori_loop(0, MS // CH, chunk, tuple([big] * NV), unroll=False)

    for k in range(NV):
        s2 = sxs[k] * sxs[k] + sys_[k] * sys_[k] + szs[k] * szs[k]
        out_v[pl.ds(k * L, L)] = accs[k] + s2
    pltpu.sync_copy(out_v, out_hbm.at[wid])


ST_TC = 512
TT_TC = 1280
KP = 128


def _tc_body(s_ref, t_ref, o_ref):
    ti = pl.program_id(1)
    s = s_ref[...]
    t = t_ref[...]
    g = jax.lax.dot_general(
        s, t, (((1,), (1,)), ((), ())),
        preferred_element_type=jnp.float32)
    s2 = jnp.sum(s * s, axis=1, keepdims=True)
    t2 = jnp.sum(t * t, axis=1, keepdims=True)
    d2 = s2 + (t2.T - jnp.float32(2.0) * g)
    m = jnp.min(d2, axis=1, keepdims=True)

    @pl.when(ti == 0)
    def _init():
        o_ref[...] = m

    @pl.when(ti != 0)
    def _acc():
        o_ref[...] = jnp.minimum(o_ref[...], m)


_nn_partials_tc = pl.pallas_call(
    _tc_body,
    grid=(N // ST_TC, MT // TT_TC),
    in_specs=[
        pl.BlockSpec((ST_TC, KP), lambda si, ti: (si, 0)),
        pl.BlockSpec((TT_TC, KP), lambda si, ti: (ti, 0)),
    ],
    out_specs=pl.BlockSpec((ST_TC, 1), lambda si, ti: (si, 0)),
    out_shape=jax.ShapeDtypeStruct((N, 1), jnp.float32),
    compiler_params=pltpu.CompilerParams(
        dimension_semantics=("parallel", "arbitrary")),
)


def kernel(source, target):
    st = source.T
    tt = target.T
    sc_out = _nn_partials_sc(st[0], st[1], st[2],
                             tt[0, :MS], tt[1, :MS], tt[2, :MS])
    s_pad = jnp.pad(source, ((0, 0), (0, KP - 3)))
    t_pad = jnp.pad(target[MS:], ((0, 0), (0, KP - 3)))
    tc_out = _nn_partials_tc(s_pad, t_pad)
    mind = jnp.minimum(sc_out.reshape(N), tc_out.reshape(N))
    rmse = jnp.sqrt(jnp.sum(mind) / jnp.float32(N))
    return jnp.where(rmse < jnp.float32(0.1), jnp.float32(1.0),
                     jnp.float32(0.0))
